# baseline (speedup 1.0000x reference)
import functools

import jax
import jax.numpy as jnp
from jax import lax
from jax.experimental import pallas as pl
from jax.experimental.pallas import tpu as pltpu
from jax.experimental.pallas import tpu_sc as plsc

B = 16384
D = 64
L = 16

_info = plsc.get_sparse_core_info()
_NC = _info.num_cores
_NW = _info.num_cores * _info.num_subcores
BPW = B // _NW
NBLK = BPW // L


def _body(da_hbm, db_hbm, emb_hbm, bias_hbm, w_hbm, b0_hbm, out_hbm,
          idx_a, idx_b, rows_a, rows_b, bia, bib, w_v, b0_v, out_v, sem):
    wid = lax.axis_index("s") * _NC + lax.axis_index("c")
    base = wid * BPW

    pltpu.sync_copy(da_hbm.at[pl.ds(base, BPW)], idx_a)
    pltpu.sync_copy(db_hbm.at[pl.ds(base, BPW)], idx_b)
    pltpu.sync_copy(w_hbm, w_v)
    pltpu.sync_copy(b0_hbm, b0_v)

    pltpu.async_copy(emb_hbm.at[idx_a], rows_a, sem).wait()
    pltpu.async_copy(emb_hbm.at[idx_b], rows_b, sem).wait()
    pltpu.async_copy(bias_hbm.at[idx_a], bia, sem).wait()
    pltpu.async_copy(bias_hbm.at[idx_b], bib, sem).wait()

    lane = jnp.arange(L, dtype=jnp.int32)

    def _splat(vec, j):
        idx = jnp.full((L, 1), j, jnp.int32)
        dnums = lax.GatherDimensionNumbers(
            offset_dims=(), collapsed_slice_dims=(0,), start_index_map=(0,))
        return lax.gather(vec, idx, dnums, (1,),
                          mode=lax.GatherScatterMode.PROMISE_IN_BOUNDS)

    wchunks = [w_v[pl.ds(c * L, L)] for c in range(D // L)]
    wtail = w_v[pl.ds(D, L)]
    w_last = _splat(wtail, 0)
    b0_vec = _splat(b0_v[pl.ds(0, L)], 0)

    def blk_body(blk, carry):
        off = pl.multiple_of(blk * L, L)
        rowidx = blk * L + lane
        ba = bia[pl.ds(off, L)]
        bb = bib[pl.ds(off, L)]
        acc = (ba + bb) * w_last + b0_vec
        for c in range(D // L):
            for j in range(L):
                k = c * L + j
                kv = jnp.full((L,), k, jnp.int32)
                av = plsc.load_gather(rows_a, [rowidx, kv])
                bv = plsc.load_gather(rows_b, [rowidx, kv])
                acc = acc + av * bv * _splat(wchunks[c], j)
        res = 1.0 / (1.0 + jnp.exp(-acc))
        out_v[pl.ds(off, L)] = res
        return carry

    lax.fori_loop(0, NBLK, blk_body, 0)
    pltpu.sync_copy(out_v, out_hbm.at[pl.ds(base, BPW)])


@jax.jit
def _afmp(da, db, emb_table, bias_table, dense_w, dense_b):
    f = functools.partial(
        pl.kernel,
        mesh=plsc.VectorSubcoreMesh(core_axis_name="c", subcore_axis_name="s"),
        compiler_params=pltpu.CompilerParams(
            use_tc_tiling_on_sc=False, needs_layout_passes=False),
        out_type=jax.ShapeDtypeStruct((B,), jnp.float32),
        scratch_types=[
            pltpu.VMEM((BPW,), jnp.int32),
            pltpu.VMEM((BPW,), jnp.int32),
            pltpu.VMEM((BPW, D), jnp.float32),
            pltpu.VMEM((BPW, D), jnp.float32),
            pltpu.VMEM((BPW,), jnp.float32),
            pltpu.VMEM((BPW,), jnp.float32),
            pltpu.VMEM((D + L,), jnp.float32),
            pltpu.VMEM((L,), jnp.float32),
            pltpu.VMEM((BPW,), jnp.float32),
            pltpu.SemaphoreType.DMA,
        ],
    )(_body)
    return f(da, db, emb_table, bias_table, dense_w, dense_b)


def kernel(drug_a, drug_b, emb_table, bias_table, dense_w, dense_b):
    da = drug_a.astype(jnp.int32)
    db = drug_b.astype(jnp.int32)
    w_pad = jnp.pad(dense_w.reshape(-1), (0, L - 1))
    b0_pad = jnp.pad(dense_b, (0, L - 1))
    out = _afmp(da, db, emb_table, bias_table.reshape(-1), w_pad, b0_pad)
    return out.reshape(B, 1)
